# baseline (speedup 1.0000x reference)
import functools

import jax
import jax.numpy as jnp
from jax import lax
from jax.experimental import pallas as pl
from jax.experimental.pallas import tpu as pltpu
from jax.experimental.pallas import tpu_sc as plsc

F32 = jnp.float32
D = 128


def _tc1_body(x_ref, w_ref, oa_ref, ob_ref):
    y = jnp.dot(x_ref[...], w_ref[...], preferred_element_type=F32)
    oa_ref[...] = y[:, :D]
    ob_ref[...] = y[:, D:]


def _fuse1_body(ma_ref, mb_ref, ya_ref, yb_ref, dp_ref, b1_ref, ws_ref,
                wn_ref, s2_ref, p2_ref):
    m = jnp.concatenate([ma_ref[0] + ma_ref[1], mb_ref[0] + mb_ref[1]], axis=1)
    y = jnp.concatenate([ya_ref[...], yb_ref[...]], axis=1)
    deg = dp_ref[0, :, 0:1] + dp_ref[1, :, 0:1]
    r = 1.0 / (deg + 1.0)
    h1 = jnp.maximum((m + y) * r + b1_ref[...], 0.0)
    s2_ref[...] = jnp.dot(h1, ws_ref[...], preferred_element_type=F32)
    p2_ref[...] = jnp.dot(h1, wn_ref[...], preferred_element_type=F32)


def _fuse2_body(s2_ref, mp_ref, dp_ref, b2_ref, h2_ref):
    deg = dp_ref[0, :, 0:1] + dp_ref[1, :, 0:1]
    r = 1.0 / jnp.maximum(deg, 1.0)
    agg = (mp_ref[0] + mp_ref[1]) * r
    h2_ref[...] = jnp.maximum(s2_ref[...] + agg + b2_ref[...], 0.0)


def _fuse3_body(h2_ref, mp_ref, dp_ref, ws_ref, wn_ref, b3_ref, o_ref):
    deg = dp_ref[0, :, 0:1] + dp_ref[1, :, 0:1]
    r = 1.0 / jnp.maximum(deg, 1.0)
    agg = (mp_ref[0] + mp_ref[1]) * r
    o_ref[...] = (jnp.dot(h2_ref[...], ws_ref[...], preferred_element_type=F32)
                  + jnp.dot(agg, wn_ref[...], preferred_element_type=F32)
                  + b3_ref[...])


def _make_scseg(NP, NCH, CK):
    ROWS = NP // 16
    mesh = plsc.VectorSubcoreMesh(core_axis_name="c", subcore_axis_name="s")

    @functools.partial(
        pl.kernel,
        out_type=jax.ShapeDtypeStruct((2, NP, D), F32),
        mesh=mesh,
        scratch_types=[
            pltpu.VMEM((NCH, CK), jnp.int32),
            pltpu.VMEM((NCH, CK), jnp.int32),
            pltpu.VMEM((CK, D), F32),
            pltpu.VMEM_SHARED((NP, D), F32),
            pltpu.SemaphoreType.DMA,
        ],
    )
    def sck(src_hbm, dst_hbm, tab_hbm, z_hbm, out_hbm,
            src_v, dst_v, rows_v, acc, sem):
        c = lax.axis_index("c")
        s = lax.axis_index("s")
        w = s * 2 + c
        pltpu.sync_copy(src_hbm.at[w], src_v)
        pltpu.sync_copy(dst_hbm.at[w], dst_v)
        pltpu.sync_copy(z_hbm.at[pl.ds(s * ROWS, ROWS)],
                        acc.at[pl.ds(s * ROWS, ROWS)])
        plsc.subcore_barrier()

        def chunk(j, carry):
            pltpu.async_copy(tab_hbm.at[src_v.at[j]], rows_v, sem).wait()
            pltpu.sync_copy(rows_v, acc.at[dst_v.at[j]], add=True)
            return carry

        lax.fori_loop(0, NCH, chunk, 0)
        plsc.subcore_barrier()
        pltpu.sync_copy(acc.at[pl.ds(s * ROWS, ROWS)],
                        out_hbm.at[c].at[pl.ds(s * ROWS, ROWS)])

    return sck


def kernel(x, edge_index, W1n, b1, W2s, W2n, b2, W3s, W3n, b3):
    N, DIN = x.shape
    H1 = W1n.shape[1]
    H2 = W2s.shape[1]
    C = W3s.shape[1]
    E = edge_index.shape[1]

    BN = 512
    NP = ((N + BN - 1) // BN) * BN
    G = NP // BN
    CK = 128
    EP = ((E + 32 * CK - 1) // (32 * CK)) * (32 * CK)
    NCH = EP // 32 // CK

    pad = EP - E
    srcp = jnp.concatenate([edge_index[0], jnp.zeros((pad,), jnp.int32)])
    dstp = jnp.concatenate([edge_index[1], jnp.full((pad,), NP - 1, jnp.int32)])
    src32 = srcp.reshape(32, NCH, CK)
    dst32 = dstp.reshape(32, NCH, CK)
    ones_tab = jnp.ones((NP, D), F32)
    zeros = jnp.zeros((NP, D), F32)

    b1r = b1.reshape(1, H1)
    b3r = b3.reshape(1, C)
    W2s_p = jnp.pad(W2s, ((0, 0), (0, D - H2)))
    W2n_p = jnp.pad(W2n, ((0, 0), (0, D - H2)))
    b2r = jnp.pad(b2, (0, D - H2)).reshape(1, D)
    W3s_p = jnp.pad(W3s, ((0, D - H2), (0, 0)))
    W3n_p = jnp.pad(W3n, ((0, D - H2), (0, 0)))

    y1a, y1b = pl.pallas_call(
        _tc1_body,
        grid=(G,),
        in_specs=[pl.BlockSpec((BN, DIN), lambda i: (i, 0)),
                  pl.BlockSpec((DIN, H1), lambda i: (0, 0))],
        out_specs=[pl.BlockSpec((BN, D), lambda i: (i, 0)),
                   pl.BlockSpec((BN, D), lambda i: (i, 0))],
        out_shape=[jax.ShapeDtypeStruct((NP, D), F32),
                   jax.ShapeDtypeStruct((NP, D), F32)],
    )(x, W1n)

    scseg = _make_scseg(NP, NCH, CK)
    degp = scseg(src32, dst32, ones_tab, zeros)
    m1a = scseg(src32, dst32, y1a, zeros)
    m1b = scseg(src32, dst32, y1b, zeros)

    s2, p2 = pl.pallas_call(
        _fuse1_body,
        grid=(G,),
        in_specs=[pl.BlockSpec((2, BN, D), lambda i: (0, i, 0)),
                  pl.BlockSpec((2, BN, D), lambda i: (0, i, 0)),
                  pl.BlockSpec((BN, D), lambda i: (i, 0)),
                  pl.BlockSpec((BN, D), lambda i: (i, 0)),
                  pl.BlockSpec((2, BN, D), lambda i: (0, i, 0)),
                  pl.BlockSpec((1, H1), lambda i: (0, 0)),
                  pl.BlockSpec((H1, D), lambda i: (0, 0)),
                  pl.BlockSpec((H1, D), lambda i: (0, 0))],
        out_specs=[pl.BlockSpec((BN, D), lambda i: (i, 0)),
                   pl.BlockSpec((BN, D), lambda i: (i, 0))],
        out_shape=[jax.ShapeDtypeStruct((NP, D), F32),
                   jax.ShapeDtypeStruct((NP, D), F32)],
    )(m1a, m1b, y1a, y1b, degp, b1r, W2s_p, W2n_p)

    m2 = scseg(src32, dst32, p2, zeros)

    h2 = pl.pallas_call(
        _fuse2_body,
        grid=(G,),
        in_specs=[pl.BlockSpec((BN, D), lambda i: (i, 0)),
                  pl.BlockSpec((2, BN, D), lambda i: (0, i, 0)),
                  pl.BlockSpec((2, BN, D), lambda i: (0, i, 0)),
                  pl.BlockSpec((1, D), lambda i: (0, 0))],
        out_specs=pl.BlockSpec((BN, D), lambda i: (i, 0)),
        out_shape=jax.ShapeDtypeStruct((NP, D), F32),
    )(s2, m2, degp, b2r)

    m3 = scseg(src32, dst32, h2, zeros)

    out = pl.pallas_call(
        _fuse3_body,
        grid=(G,),
        in_specs=[pl.BlockSpec((BN, D), lambda i: (i, 0)),
                  pl.BlockSpec((2, BN, D), lambda i: (0, i, 0)),
                  pl.BlockSpec((2, BN, D), lambda i: (0, i, 0)),
                  pl.BlockSpec((D, C), lambda i: (0, 0)),
                  pl.BlockSpec((D, C), lambda i: (0, 0)),
                  pl.BlockSpec((1, C), lambda i: (0, 0))],
        out_specs=pl.BlockSpec((BN, C), lambda i: (i, 0)),
        out_shape=jax.ShapeDtypeStruct((N, C), F32),
    )(h2, m3, degp, W3s_p, W3n_p, b3r)

    return out


# speedup vs baseline: 1.3137x; 1.3137x over previous
import functools

import jax
import jax.numpy as jnp
from jax import lax
from jax.experimental import pallas as pl
from jax.experimental.pallas import tpu as pltpu
from jax.experimental.pallas import tpu_sc as plsc

F32 = jnp.float32
D = 128


def _tc1_body(x_ref, w_ref, oa_ref, ob_ref):
    y = jnp.dot(x_ref[...], w_ref[...], preferred_element_type=F32)
    oa_ref[...] = y[:, :D]
    ob_ref[...] = y[:, D:]


def _fuse1_body(ma_ref, mb_ref, ya_ref, yb_ref, dp_ref, b1_ref, ws_ref,
                wn_ref, s2_ref, p2_ref):
    m = jnp.concatenate([ma_ref[0] + ma_ref[1], mb_ref[0] + mb_ref[1]], axis=1)
    y = jnp.concatenate([ya_ref[...], yb_ref[...]], axis=1)
    deg = dp_ref[0, :, 0:1] + dp_ref[1, :, 0:1]
    r = 1.0 / (deg + 1.0)
    h1 = jnp.maximum((m + y) * r + b1_ref[...], 0.0)
    s2_ref[...] = jnp.dot(h1, ws_ref[...], preferred_element_type=F32)
    p2_ref[...] = jnp.dot(h1, wn_ref[...], preferred_element_type=F32)


def _fuse2_body(s2_ref, mp_ref, dp_ref, b2_ref, h2_ref):
    deg = dp_ref[0, :, 0:1] + dp_ref[1, :, 0:1]
    r = 1.0 / jnp.maximum(deg, 1.0)
    agg = (mp_ref[0] + mp_ref[1]) * r
    h2_ref[...] = jnp.maximum(s2_ref[...] + agg + b2_ref[...], 0.0)


def _fuse3_body(h2_ref, mp_ref, dp_ref, ws_ref, wn_ref, b3_ref, o_ref):
    deg = dp_ref[0, :, 0:1] + dp_ref[1, :, 0:1]
    r = 1.0 / jnp.maximum(deg, 1.0)
    agg = (mp_ref[0] + mp_ref[1]) * r
    o_ref[...] = (jnp.dot(h2_ref[...], ws_ref[...], preferred_element_type=F32)
                  + jnp.dot(agg, wn_ref[...], preferred_element_type=F32)
                  + b3_ref[...])


def _make_scseg(NP, NCH, CK):
    ROWS = NP // 16
    mesh = plsc.VectorSubcoreMesh(core_axis_name="c", subcore_axis_name="s")

    @functools.partial(
        pl.kernel,
        out_type=jax.ShapeDtypeStruct((2, NP, D), F32),
        mesh=mesh,
        scratch_types=[
            pltpu.VMEM((NCH, CK), jnp.int32),
            pltpu.VMEM((NCH, CK), jnp.int32),
            pltpu.VMEM((CK, D), F32),
            pltpu.VMEM_SHARED((NP, D), F32),
            pltpu.SemaphoreType.DMA,
        ],
    )
    def sck(src_hbm, dst_hbm, tab_hbm, z_hbm, out_hbm,
            src_v, dst_v, rows_v, acc, sem):
        c = lax.axis_index("c")
        s = lax.axis_index("s")
        w = s * 2 + c
        pltpu.sync_copy(src_hbm.at[w], src_v)
        pltpu.sync_copy(dst_hbm.at[w], dst_v)
        pltpu.sync_copy(z_hbm.at[pl.ds(s * ROWS, ROWS)],
                        acc.at[pl.ds(s * ROWS, ROWS)])
        plsc.subcore_barrier()

        def chunk(j, carry):
            pltpu.async_copy(tab_hbm.at[src_v.at[j]], rows_v, sem).wait()
            pltpu.sync_copy(rows_v, acc.at[dst_v.at[j]], add=True)
            return carry

        lax.fori_loop(0, NCH, chunk, 0)
        plsc.subcore_barrier()
        pltpu.sync_copy(acc.at[pl.ds(s * ROWS, ROWS)],
                        out_hbm.at[c].at[pl.ds(s * ROWS, ROWS)])

    return sck


def kernel(x, edge_index, W1n, b1, W2s, W2n, b2, W3s, W3n, b3):
    N, DIN = x.shape
    H1 = W1n.shape[1]
    H2 = W2s.shape[1]
    C = W3s.shape[1]
    E = edge_index.shape[1]

    BN = 512
    NP = ((N + BN - 1) // BN) * BN
    G = NP // BN
    CK = 64
    EP = ((E + 32 * CK - 1) // (32 * CK)) * (32 * CK)
    NCH = EP // 32 // CK

    pad = EP - E
    srcp = jnp.concatenate([edge_index[0], jnp.zeros((pad,), jnp.int32)])
    dstp = jnp.concatenate([edge_index[1], jnp.full((pad,), NP - 1, jnp.int32)])
    src32 = srcp.reshape(32, NCH, CK)
    dst32 = dstp.reshape(32, NCH, CK)
    ones_tab = jnp.ones((NP, D), F32)
    zeros = jnp.zeros((NP, D), F32)

    b1r = b1.reshape(1, H1)
    b3r = b3.reshape(1, C)
    W2s_p = jnp.pad(W2s, ((0, 0), (0, D - H2)))
    W2n_p = jnp.pad(W2n, ((0, 0), (0, D - H2)))
    b2r = jnp.pad(b2, (0, D - H2)).reshape(1, D)
    W3s_p = jnp.pad(W3s, ((0, D - H2), (0, 0)))
    W3n_p = jnp.pad(W3n, ((0, D - H2), (0, 0)))

    y1a, y1b = pl.pallas_call(
        _tc1_body,
        grid=(G,),
        in_specs=[pl.BlockSpec((BN, DIN), lambda i: (i, 0)),
                  pl.BlockSpec((DIN, H1), lambda i: (0, 0))],
        out_specs=[pl.BlockSpec((BN, D), lambda i: (i, 0)),
                   pl.BlockSpec((BN, D), lambda i: (i, 0))],
        out_shape=[jax.ShapeDtypeStruct((NP, D), F32),
                   jax.ShapeDtypeStruct((NP, D), F32)],
    )(x, W1n)

    scseg = _make_scseg(NP, NCH, CK)
    degp = scseg(src32, dst32, ones_tab, zeros)
    m1a = scseg(src32, dst32, y1a, zeros)
    m1b = scseg(src32, dst32, y1b, zeros)

    s2, p2 = pl.pallas_call(
        _fuse1_body,
        grid=(G,),
        in_specs=[pl.BlockSpec((2, BN, D), lambda i: (0, i, 0)),
                  pl.BlockSpec((2, BN, D), lambda i: (0, i, 0)),
                  pl.BlockSpec((BN, D), lambda i: (i, 0)),
                  pl.BlockSpec((BN, D), lambda i: (i, 0)),
                  pl.BlockSpec((2, BN, D), lambda i: (0, i, 0)),
                  pl.BlockSpec((1, H1), lambda i: (0, 0)),
                  pl.BlockSpec((H1, D), lambda i: (0, 0)),
                  pl.BlockSpec((H1, D), lambda i: (0, 0))],
        out_specs=[pl.BlockSpec((BN, D), lambda i: (i, 0)),
                   pl.BlockSpec((BN, D), lambda i: (i, 0))],
        out_shape=[jax.ShapeDtypeStruct((NP, D), F32),
                   jax.ShapeDtypeStruct((NP, D), F32)],
    )(m1a, m1b, y1a, y1b, degp, b1r, W2s_p, W2n_p)

    m2 = scseg(src32, dst32, p2, zeros)

    h2 = pl.pallas_call(
        _fuse2_body,
        grid=(G,),
        in_specs=[pl.BlockSpec((BN, D), lambda i: (i, 0)),
                  pl.BlockSpec((2, BN, D), lambda i: (0, i, 0)),
                  pl.BlockSpec((2, BN, D), lambda i: (0, i, 0)),
                  pl.BlockSpec((1, D), lambda i: (0, 0))],
        out_specs=pl.BlockSpec((BN, D), lambda i: (i, 0)),
        out_shape=jax.ShapeDtypeStruct((NP, D), F32),
    )(s2, m2, degp, b2r)

    m3 = scseg(src32, dst32, h2, zeros)

    out = pl.pallas_call(
        _fuse3_body,
        grid=(G,),
        in_specs=[pl.BlockSpec((BN, D), lambda i: (i, 0)),
                  pl.BlockSpec((2, BN, D), lambda i: (0, i, 0)),
                  pl.BlockSpec((2, BN, D), lambda i: (0, i, 0)),
                  pl.BlockSpec((D, C), lambda i: (0, 0)),
                  pl.BlockSpec((D, C), lambda i: (0, 0)),
                  pl.BlockSpec((1, C), lambda i: (0, 0))],
        out_specs=pl.BlockSpec((BN, C), lambda i: (i, 0)),
        out_shape=jax.ShapeDtypeStruct((N, C), F32),
    )(h2, m3, degp, W3s_p, W3n_p, b3r)

    return out


# speedup vs baseline: 1.5830x; 1.2050x over previous
import functools

import jax
import jax.numpy as jnp
from jax import lax
from jax.experimental import pallas as pl
from jax.experimental.pallas import tpu as pltpu
from jax.experimental.pallas import tpu_sc as plsc

F32 = jnp.float32
D = 128


def _tc1_body(x_ref, w_ref, oa_ref, ob_ref):
    y = jnp.dot(x_ref[...], w_ref[...], preferred_element_type=F32)
    oa_ref[...] = y[:, :D]
    ob_ref[...] = y[:, D:]


def _fuse1_body(ma_ref, mb_ref, ya_ref, yb_ref, dp_ref, b1_ref, ws_ref,
                wn_ref, s2_ref, p2_ref):
    m = jnp.concatenate([ma_ref[0] + ma_ref[1], mb_ref[0] + mb_ref[1]], axis=1)
    y = jnp.concatenate([ya_ref[...], yb_ref[...]], axis=1)
    deg = dp_ref[0, :, 0:1] + dp_ref[1, :, 0:1]
    r = 1.0 / (deg + 1.0)
    h1 = jnp.maximum((m + y) * r + b1_ref[...], 0.0)
    s2_ref[...] = jnp.dot(h1, ws_ref[...], preferred_element_type=F32)
    p2_ref[...] = jnp.dot(h1, wn_ref[...], preferred_element_type=F32)


def _fuse2_body(s2_ref, mp_ref, dp_ref, b2_ref, h2_ref):
    deg = dp_ref[0, :, 0:1] + dp_ref[1, :, 0:1]
    r = 1.0 / jnp.maximum(deg, 1.0)
    agg = (mp_ref[0] + mp_ref[1]) * r
    h2_ref[...] = jnp.maximum(s2_ref[...] + agg + b2_ref[...], 0.0)


def _fuse3_body(h2_ref, mp_ref, dp_ref, ws_ref, wn_ref, b3_ref, o_ref):
    deg = dp_ref[0, :, 0:1] + dp_ref[1, :, 0:1]
    r = 1.0 / jnp.maximum(deg, 1.0)
    agg = (mp_ref[0] + mp_ref[1]) * r
    o_ref[...] = (jnp.dot(h2_ref[...], ws_ref[...], preferred_element_type=F32)
                  + jnp.dot(agg, wn_ref[...], preferred_element_type=F32)
                  + b3_ref[...])


def _make_scseg(NP, NCH, CK):
    ROWS = NP // 16
    mesh = plsc.VectorSubcoreMesh(core_axis_name="c", subcore_axis_name="s")

    @functools.partial(
        pl.kernel,
        out_type=jax.ShapeDtypeStruct((2, NP, D), F32),
        mesh=mesh,
        scratch_types=[
            pltpu.VMEM((NCH, CK), jnp.int32),
            pltpu.VMEM((NCH, CK), jnp.int32),
            pltpu.VMEM((CK, D), F32),
            pltpu.VMEM_SHARED((NP, D), F32),
            pltpu.SemaphoreType.DMA,
        ],
    )
    def sck(src_hbm, dst_hbm, tab_hbm, z_hbm, out_hbm,
            src_v, dst_v, rows_v, acc, sem):
        c = lax.axis_index("c")
        s = lax.axis_index("s")
        w = s * 2 + c
        pltpu.sync_copy(src_hbm.at[w], src_v)
        pltpu.sync_copy(dst_hbm.at[w], dst_v)
        pltpu.sync_copy(z_hbm.at[pl.ds(s * ROWS, ROWS)],
                        acc.at[pl.ds(s * ROWS, ROWS)])
        plsc.subcore_barrier()

        def chunk(j, carry):
            pltpu.async_copy(tab_hbm.at[src_v.at[j]], rows_v, sem).wait()
            pltpu.sync_copy(rows_v, acc.at[dst_v.at[j]], add=True)
            return carry

        lax.fori_loop(0, NCH, chunk, 0)
        plsc.subcore_barrier()
        pltpu.sync_copy(acc.at[pl.ds(s * ROWS, ROWS)],
                        out_hbm.at[c].at[pl.ds(s * ROWS, ROWS)])

    return sck


def kernel(x, edge_index, W1n, b1, W2s, W2n, b2, W3s, W3n, b3):
    N, DIN = x.shape
    H1 = W1n.shape[1]
    H2 = W2s.shape[1]
    C = W3s.shape[1]
    E = edge_index.shape[1]

    BN = 512
    NP = ((N + BN - 1) // BN) * BN
    G = NP // BN
    CK = 80
    EP = ((E + 32 * CK - 1) // (32 * CK)) * (32 * CK)
    NCH = EP // 32 // CK

    pad = EP - E
    srcp = jnp.concatenate([edge_index[0], jnp.zeros((pad,), jnp.int32)])
    dstp = jnp.concatenate([edge_index[1], jnp.full((pad,), NP - 1, jnp.int32)])
    src32 = srcp.reshape(32, NCH, CK)
    dst32 = dstp.reshape(32, NCH, CK)
    ones_tab = jnp.ones((NP, D), F32)
    zeros = jnp.zeros((NP, D), F32)

    b1r = b1.reshape(1, H1)
    b3r = b3.reshape(1, C)
    W2s_p = jnp.pad(W2s, ((0, 0), (0, D - H2)))
    W2n_p = jnp.pad(W2n, ((0, 0), (0, D - H2)))
    b2r = jnp.pad(b2, (0, D - H2)).reshape(1, D)
    W3s_p = jnp.pad(W3s, ((0, D - H2), (0, 0)))
    W3n_p = jnp.pad(W3n, ((0, D - H2), (0, 0)))

    y1a, y1b = pl.pallas_call(
        _tc1_body,
        grid=(G,),
        in_specs=[pl.BlockSpec((BN, DIN), lambda i: (i, 0)),
                  pl.BlockSpec((DIN, H1), lambda i: (0, 0))],
        out_specs=[pl.BlockSpec((BN, D), lambda i: (i, 0)),
                   pl.BlockSpec((BN, D), lambda i: (i, 0))],
        out_shape=[jax.ShapeDtypeStruct((NP, D), F32),
                   jax.ShapeDtypeStruct((NP, D), F32)],
    )(x, W1n)

    scseg = _make_scseg(NP, NCH, CK)
    degp = scseg(src32, dst32, ones_tab, zeros)
    m1a = scseg(src32, dst32, y1a, zeros)
    m1b = scseg(src32, dst32, y1b, zeros)

    s2, p2 = pl.pallas_call(
        _fuse1_body,
        grid=(G,),
        in_specs=[pl.BlockSpec((2, BN, D), lambda i: (0, i, 0)),
                  pl.BlockSpec((2, BN, D), lambda i: (0, i, 0)),
                  pl.BlockSpec((BN, D), lambda i: (i, 0)),
                  pl.BlockSpec((BN, D), lambda i: (i, 0)),
                  pl.BlockSpec((2, BN, D), lambda i: (0, i, 0)),
                  pl.BlockSpec((1, H1), lambda i: (0, 0)),
                  pl.BlockSpec((H1, D), lambda i: (0, 0)),
                  pl.BlockSpec((H1, D), lambda i: (0, 0))],
        out_specs=[pl.BlockSpec((BN, D), lambda i: (i, 0)),
                   pl.BlockSpec((BN, D), lambda i: (i, 0))],
        out_shape=[jax.ShapeDtypeStruct((NP, D), F32),
                   jax.ShapeDtypeStruct((NP, D), F32)],
    )(m1a, m1b, y1a, y1b, degp, b1r, W2s_p, W2n_p)

    m2 = scseg(src32, dst32, p2, zeros)

    h2 = pl.pallas_call(
        _fuse2_body,
        grid=(G,),
        in_specs=[pl.BlockSpec((BN, D), lambda i: (i, 0)),
                  pl.BlockSpec((2, BN, D), lambda i: (0, i, 0)),
                  pl.BlockSpec((2, BN, D), lambda i: (0, i, 0)),
                  pl.BlockSpec((1, D), lambda i: (0, 0))],
        out_specs=pl.BlockSpec((BN, D), lambda i: (i, 0)),
        out_shape=jax.ShapeDtypeStruct((NP, D), F32),
    )(s2, m2, degp, b2r)

    m3 = scseg(src32, dst32, h2, zeros)

    out = pl.pallas_call(
        _fuse3_body,
        grid=(G,),
        in_specs=[pl.BlockSpec((BN, D), lambda i: (i, 0)),
                  pl.BlockSpec((2, BN, D), lambda i: (0, i, 0)),
                  pl.BlockSpec((2, BN, D), lambda i: (0, i, 0)),
                  pl.BlockSpec((D, C), lambda i: (0, 0)),
                  pl.BlockSpec((D, C), lambda i: (0, 0)),
                  pl.BlockSpec((1, C), lambda i: (0, 0))],
        out_specs=pl.BlockSpec((BN, C), lambda i: (i, 0)),
        out_shape=jax.ShapeDtypeStruct((N, C), F32),
    )(h2, m3, degp, W3s_p, W3n_p, b3r)

    return out
